# baseline (device time: 60528 ns/iter reference)
import jax
import jax.numpy as jnp
from jax import lax
from jax.experimental import pallas as pl
from jax.experimental.pallas import tpu as pltpu

N_DEV = 16
LOG2_N = 4
N_LAYERS = 3


def kernel(x, Win0, Wout0, Win1, Wout1, Win2, Wout2):
    b, _ = x.shape
    hdim = Win0.shape[1]
    out_cols = Wout0.shape[1]

    def body(x_ref, win0_ref, wout0_ref, win1_ref, wout1_ref, win2_ref,
             wout2_ref, out_ref, accum_ref, recv_ref, send_sems, recv_sems):
        my = lax.axis_index("i")
        wins = [win0_ref, win1_ref, win2_ref]
        wouts = [wout0_ref, wout1_ref, wout2_ref]

        xv = x_ref[...].astype(jnp.bfloat16)
        for layer in range(N_LAYERS):
            w_in = wins[layer][...].astype(jnp.bfloat16)
            accum_ref[...] = jnp.dot(xv, w_in, preferred_element_type=jnp.float32)

            for r in range(LOG2_N):
                partner = my ^ (1 << r)
                slot = layer * LOG2_N + r
                rdma = pltpu.make_async_remote_copy(
                    src_ref=accum_ref,
                    dst_ref=recv_ref.at[slot],
                    send_sem=send_sems.at[slot],
                    recv_sem=recv_sems.at[slot],
                    device_id=(partner,),
                    device_id_type=pl.DeviceIdType.MESH,
                )
                rdma.start()
                rdma.wait()
                accum_ref[...] = accum_ref[...] + recv_ref[slot]

            h = jnp.maximum(accum_ref[...], 0.0).astype(jnp.bfloat16)
            w_out = wouts[layer][...].astype(jnp.bfloat16)
            nxt = jnp.dot(h, w_out, preferred_element_type=jnp.float32)
            if layer < N_LAYERS - 1:
                xv = nxt.astype(jnp.bfloat16)
            else:
                out_ref[...] = nxt

    n_slots = N_LAYERS * LOG2_N
    return pl.pallas_call(
        body,
        out_shape=jax.ShapeDtypeStruct((b, out_cols), jnp.float32),
        in_specs=[pl.BlockSpec(memory_space=pltpu.VMEM)] * 7,
        out_specs=pl.BlockSpec(memory_space=pltpu.VMEM),
        scratch_shapes=[
            pltpu.VMEM((b, hdim), jnp.float32),
            pltpu.VMEM((n_slots, b, hdim), jnp.float32),
            pltpu.SemaphoreType.DMA((n_slots,)),
            pltpu.SemaphoreType.DMA((n_slots,)),
        ],
    )(x, Win0, Wout0, Win1, Wout1, Win2, Wout2)


# device time: 50438 ns/iter; 1.2000x vs baseline; 1.2000x over previous
import jax
import jax.numpy as jnp
from jax import lax
from jax.experimental import pallas as pl
from jax.experimental.pallas import tpu as pltpu

N_DEV = 16
LOG2_N = 4
N_LAYERS = 3


def kernel(x, Win0, Wout0, Win1, Wout1, Win2, Wout2):
    b, _ = x.shape
    hdim = Win0.shape[1]
    out_cols = Wout0.shape[1]

    def body(x_ref, win0_ref, wout0_ref, win1_ref, wout1_ref, win2_ref,
             wout2_ref, out_ref, accum_ref, send_ref, recv_ref,
             send_sems, recv_sems):
        my = lax.axis_index("i")
        wins = [win0_ref, win1_ref, win2_ref]
        wouts = [wout0_ref, wout1_ref, wout2_ref]

        xv = x_ref[...].astype(jnp.bfloat16)
        for layer in range(N_LAYERS):
            w_in = wins[layer][...].astype(jnp.bfloat16)
            accum_ref[...] = jnp.dot(xv, w_in, preferred_element_type=jnp.float32)

            for r in range(LOG2_N):
                partner = my ^ (1 << r)
                slot = layer * LOG2_N + r
                send_ref[...] = accum_ref[...].astype(jnp.bfloat16)
                rdma = pltpu.make_async_remote_copy(
                    src_ref=send_ref,
                    dst_ref=recv_ref.at[slot],
                    send_sem=send_sems.at[slot],
                    recv_sem=recv_sems.at[slot],
                    device_id=(partner,),
                    device_id_type=pl.DeviceIdType.MESH,
                )
                rdma.start()
                rdma.wait()
                accum_ref[...] = accum_ref[...] + recv_ref[slot].astype(jnp.float32)

            h = jnp.maximum(accum_ref[...], 0.0).astype(jnp.bfloat16)
            w_out = wouts[layer][...].astype(jnp.bfloat16)
            nxt = jnp.dot(h, w_out, preferred_element_type=jnp.float32)
            if layer < N_LAYERS - 1:
                xv = nxt.astype(jnp.bfloat16)
            else:
                out_ref[...] = nxt

    n_slots = N_LAYERS * LOG2_N
    return pl.pallas_call(
        body,
        out_shape=jax.ShapeDtypeStruct((b, out_cols), jnp.float32),
        in_specs=[pl.BlockSpec(memory_space=pltpu.VMEM)] * 7,
        out_specs=pl.BlockSpec(memory_space=pltpu.VMEM),
        scratch_shapes=[
            pltpu.VMEM((b, hdim), jnp.float32),
            pltpu.VMEM((b, hdim), jnp.bfloat16),
            pltpu.VMEM((n_slots, b, hdim), jnp.bfloat16),
            pltpu.SemaphoreType.DMA((n_slots,)),
            pltpu.SemaphoreType.DMA((n_slots,)),
        ],
    )(x, Win0, Wout0, Win1, Wout1, Win2, Wout2)


# device time: 37464 ns/iter; 1.6156x vs baseline; 1.3463x over previous
import jax
import jax.numpy as jnp
from jax import lax
from jax.experimental import pallas as pl
from jax.experimental.pallas import tpu as pltpu

N_DEV = 16
N_ROUNDS = 2
RADIX = 4
N_LAYERS = 3


def kernel(x, Win0, Wout0, Win1, Wout1, Win2, Wout2):
    b, _ = x.shape
    hdim = Win0.shape[1]
    out_cols = Wout0.shape[1]

    def body(x_ref, win0_ref, wout0_ref, win1_ref, wout1_ref, win2_ref,
             wout2_ref, out_ref, accum_ref, recv_ref,
             send_sems, recv_sems):
        my = lax.axis_index("i")
        wins = [win0_ref, win1_ref, win2_ref]
        wouts = [wout0_ref, wout1_ref, wout2_ref]

        barrier_sem = pltpu.get_barrier_semaphore()
        for r in range(N_ROUNDS):
            for j in range(RADIX - 1):
                partner = my ^ ((j + 1) << (2 * r))
                pl.semaphore_signal(
                    barrier_sem, inc=1,
                    device_id=(partner,), device_id_type=pl.DeviceIdType.MESH,
                )
        pl.semaphore_wait(barrier_sem, N_ROUNDS * (RADIX - 1))

        xv = x_ref[...].astype(jnp.bfloat16)
        for layer in range(N_LAYERS):
            w_in = wins[layer][...].astype(jnp.bfloat16)
            accum_ref[...] = jnp.dot(
                xv, w_in, preferred_element_type=jnp.float32
            ).astype(jnp.bfloat16)

            for r in range(N_ROUNDS):
                rdmas = []
                for j in range(RADIX - 1):
                    partner = my ^ ((j + 1) << (2 * r))
                    slot = (layer * N_ROUNDS + r) * (RADIX - 1) + j
                    rdma = pltpu.make_async_remote_copy(
                        src_ref=accum_ref,
                        dst_ref=recv_ref.at[slot],
                        send_sem=send_sems.at[slot],
                        recv_sem=recv_sems.at[slot],
                        device_id=(partner,),
                        device_id_type=pl.DeviceIdType.MESH,
                    )
                    rdma.start()
                    rdmas.append(rdma)
                base = (layer * N_ROUNDS + r) * (RADIX - 1)
                total = accum_ref[...].astype(jnp.float32)
                for j, rdma in enumerate(rdmas):
                    rdma.wait()
                    total = total + recv_ref[base + j].astype(jnp.float32)
                accum_ref[...] = total.astype(jnp.bfloat16)

            h = jnp.maximum(accum_ref[...], jnp.bfloat16(0.0))
            w_out = wouts[layer][...].astype(jnp.bfloat16)
            nxt = jnp.dot(h, w_out, preferred_element_type=jnp.float32)
            if layer < N_LAYERS - 1:
                xv = nxt.astype(jnp.bfloat16)
            else:
                out_ref[...] = nxt

    n_slots = N_LAYERS * N_ROUNDS * (RADIX - 1)
    return pl.pallas_call(
        body,
        out_shape=jax.ShapeDtypeStruct((b, out_cols), jnp.float32),
        in_specs=[pl.BlockSpec(memory_space=pltpu.VMEM)] * 7,
        out_specs=pl.BlockSpec(memory_space=pltpu.VMEM),
        scratch_shapes=[
            pltpu.VMEM((b, hdim), jnp.bfloat16),
            pltpu.VMEM((n_slots, b, hdim), jnp.bfloat16),
            pltpu.SemaphoreType.DMA((n_slots,)),
            pltpu.SemaphoreType.DMA((n_slots,)),
        ],
        compiler_params=pltpu.CompilerParams(collective_id=0),
    )(x, Win0, Wout0, Win1, Wout1, Win2, Wout2)
